# baseline (device time: 13101 ns/iter reference)
import jax
import jax.numpy as jnp
from jax import lax
from jax.experimental import pallas as pl
from jax.experimental.pallas import tpu as pltpu

N_DEV = 8
N_TOK = 512
D_IN = 256
D_OUT = 512
E_LOC = 2
CAP = 25
CHUNK = N_TOK // N_DEV


def kernel(x, router_W, route_idx, expert_W):
    del router_W

    def body(x_ref, idx_ref, w_ref, out_ref, xm_ref, send_ref,
             comm_ref, send_sems, recv_sems):
        my_id = lax.axis_index("i")

        barrier_sem = pltpu.get_barrier_semaphore()
        for off in range(1, N_DEV):
            dst = lax.rem(my_id + off, N_DEV)
            pl.semaphore_signal(barrier_sem, inc=1, device_id=(dst,),
                                device_id_type=pl.DeviceIdType.MESH)

        e_row = lax.broadcasted_iota(jnp.int32, (1, E_LOC), 1) + E_LOC * my_id
        onehot = idx_ref[:, :] == e_row
        r = lax.broadcasted_iota(jnp.int32, (N_TOK, N_TOK), 0)
        c = lax.broadcasted_iota(jnp.int32, (N_TOK, N_TOK), 1)
        tril = (c <= r).astype(jnp.float32)
        cnt = jnp.dot(tril, onehot.astype(jnp.float32),
                      preferred_element_type=jnp.float32)
        m = jnp.where(onehot & (cnt <= CAP), 1.0, 0.0)

        xi = x_ref[:, :]
        xm = jnp.concatenate([xi * m[:, 0:1], xi * m[:, 1:2]], axis=1)
        xm_ref[:, :, :] = xm.reshape(N_DEV, CHUNK, E_LOC * D_IN)
        w2 = w_ref[:, :, :].reshape(E_LOC * D_IN, D_OUT)
        for off in range(1, N_DEV):
            dst = lax.rem(my_id + off, N_DEV)
            y = jnp.dot(xm_ref[dst], w2, preferred_element_type=jnp.float32)
            send_ref[off - 1, :, :] = y.astype(jnp.bfloat16)
        acc = jnp.dot(xm_ref[my_id], w2, preferred_element_type=jnp.float32)

        pl.semaphore_wait(barrier_sem, N_DEV - 1)

        rdmas = []
        for off in range(1, N_DEV):
            dst = lax.rem(my_id + off, N_DEV)
            rdma = pltpu.make_async_remote_copy(
                src_ref=send_ref.at[off - 1],
                dst_ref=comm_ref.at[off - 1],
                send_sem=send_sems.at[off - 1],
                recv_sem=recv_sems.at[off - 1],
                device_id=(dst,),
                device_id_type=pl.DeviceIdType.MESH,
            )
            rdma.start()
            rdmas.append(rdma)

        for off in range(1, N_DEV):
            rdmas[off - 1].wait_recv()
            acc = acc + comm_ref[off - 1, :, :].astype(jnp.float32)
        for off in range(1, N_DEV):
            rdmas[off - 1].wait_send()
        out_ref[:, :] = acc

    return pl.pallas_call(
        body,
        out_shape=jax.ShapeDtypeStruct((CHUNK, D_OUT), jnp.float32),
        in_specs=[
            pl.BlockSpec(memory_space=pltpu.VMEM),
            pl.BlockSpec(memory_space=pltpu.VMEM),
            pl.BlockSpec(memory_space=pltpu.VMEM),
        ],
        out_specs=pl.BlockSpec(memory_space=pltpu.VMEM),
        scratch_shapes=[
            pltpu.VMEM((N_DEV, CHUNK, E_LOC * D_IN), jnp.float32),
            pltpu.VMEM((N_DEV - 1, CHUNK, D_OUT), jnp.bfloat16),
            pltpu.VMEM((N_DEV - 1, CHUNK, D_OUT), jnp.bfloat16),
            pltpu.SemaphoreType.DMA((N_DEV - 1,)),
            pltpu.SemaphoreType.DMA((N_DEV - 1,)),
        ],
        compiler_params=pltpu.CompilerParams(collective_id=0),
    )(x, route_idx, expert_W)


# device time: 12059 ns/iter; 1.0864x vs baseline; 1.0864x over previous
import jax
import jax.numpy as jnp
from jax import lax
from jax.experimental import pallas as pl
from jax.experimental.pallas import tpu as pltpu

N_DEV = 8
N_TOK = 512
D_IN = 256
D_OUT = 512
E_LOC = 2
CAP = 25
CHUNK = N_TOK // N_DEV


def kernel(x, router_W, route_idx, expert_W):
    del router_W

    xb = x.astype(jnp.bfloat16)
    wb = expert_W.astype(jnp.bfloat16)

    def body(x_ref, idx_ref, w_ref, out_ref, xm_ref, send_ref,
             comm_ref, send_sems, recv_sems):
        my_id = lax.axis_index("i")

        barrier_sem = pltpu.get_barrier_semaphore()
        for off in range(1, N_DEV):
            dst = lax.rem(my_id + off, N_DEV)
            pl.semaphore_signal(barrier_sem, inc=1, device_id=(dst,),
                                device_id_type=pl.DeviceIdType.MESH)

        e_row = lax.broadcasted_iota(jnp.int32, (1, E_LOC), 1) + E_LOC * my_id
        onehot = idx_ref[:, :] == e_row
        r = lax.broadcasted_iota(jnp.int32, (N_TOK, N_TOK), 0)
        c = lax.broadcasted_iota(jnp.int32, (N_TOK, N_TOK), 1)
        tril = (c <= r).astype(jnp.float32)
        cnt = jnp.dot(tril, onehot.astype(jnp.float32),
                      preferred_element_type=jnp.float32)
        m = jnp.where(onehot & (cnt <= CAP), 1.0, 0.0).astype(jnp.bfloat16)

        xi = x_ref[:, :]
        xm = jnp.concatenate([xi * m[:, 0:1], xi * m[:, 1:2]], axis=1)
        xm_ref[:, :, :] = xm.reshape(N_DEV, CHUNK, E_LOC * D_IN)
        w2 = w_ref[:, :, :].reshape(E_LOC * D_IN, D_OUT)
        for off in range(1, N_DEV):
            dst = lax.rem(my_id + off, N_DEV)
            y = jnp.dot(xm_ref[dst], w2, preferred_element_type=jnp.float32)
            send_ref[off - 1, :, :] = y.astype(jnp.bfloat16)
        acc = jnp.dot(xm_ref[my_id], w2, preferred_element_type=jnp.float32)

        pl.semaphore_wait(barrier_sem, N_DEV - 1)

        rdmas = []
        for off in range(1, N_DEV):
            dst = lax.rem(my_id + off, N_DEV)
            rdma = pltpu.make_async_remote_copy(
                src_ref=send_ref.at[off - 1],
                dst_ref=comm_ref.at[off - 1],
                send_sem=send_sems.at[off - 1],
                recv_sem=recv_sems.at[off - 1],
                device_id=(dst,),
                device_id_type=pl.DeviceIdType.MESH,
            )
            rdma.start()
            rdmas.append(rdma)

        for off in range(1, N_DEV):
            rdmas[off - 1].wait_recv()
            acc = acc + comm_ref[off - 1, :, :].astype(jnp.float32)
        for off in range(1, N_DEV):
            rdmas[off - 1].wait_send()
        out_ref[:, :] = acc

    return pl.pallas_call(
        body,
        out_shape=jax.ShapeDtypeStruct((CHUNK, D_OUT), jnp.float32),
        in_specs=[
            pl.BlockSpec(memory_space=pltpu.VMEM),
            pl.BlockSpec(memory_space=pltpu.VMEM),
            pl.BlockSpec(memory_space=pltpu.VMEM),
        ],
        out_specs=pl.BlockSpec(memory_space=pltpu.VMEM),
        scratch_shapes=[
            pltpu.VMEM((N_DEV, CHUNK, E_LOC * D_IN), jnp.bfloat16),
            pltpu.VMEM((N_DEV - 1, CHUNK, D_OUT), jnp.bfloat16),
            pltpu.VMEM((N_DEV - 1, CHUNK, D_OUT), jnp.bfloat16),
            pltpu.SemaphoreType.DMA((N_DEV - 1,)),
            pltpu.SemaphoreType.DMA((N_DEV - 1,)),
        ],
        compiler_params=pltpu.CompilerParams(collective_id=0),
    )(xb, route_idx, wb)
